# baseline (speedup 1.0000x reference)
import jax
import jax.numpy as jnp
from jax import lax
from jax.experimental import pallas as pl
from jax.experimental.pallas import tpu as pltpu
from jax.experimental.pallas import tpu_sc as plsc
from jax._src.pallas import mpmd as _mpmd

_REP_DIM = 768
_HID = 384
_TOPK = 64
_TEMP = 0.1
_B, _N = 32, 2048
_BN = 1024
_NB = _N // _BN


def _logits_body(reps_ref, w1_ref, b1_ref, w2_ref, lg_ref, base_ref):
    x = reps_ref[0]
    h = jax.lax.dot_general(
        x, w1_ref[...], (((1,), (1,)), ((), ())),
        preferred_element_type=jnp.float32,
    )
    h = jnp.tanh(h + b1_ref[...])
    h = h.astype(jnp.bfloat16).astype(jnp.float32)
    w2 = w2_ref[...].astype(jnp.bfloat16).astype(jnp.float32)
    lg_ref[0, 0, 0, :] = jnp.sum(h * w2, axis=1)
    base_ref[0] = jnp.zeros((_BN, _REP_DIM), jnp.float32)


def _compute_logits_and_base(reps, W1, b1, W2):
    lg, base = pl.pallas_call(
        _logits_body,
        grid=(_B * _N // _BN,),
        in_specs=[
            pl.BlockSpec((1, _BN, _REP_DIM), lambda i: (i // _NB, i % _NB, 0)),
            pl.BlockSpec((_HID, _REP_DIM), lambda i: (0, 0)),
            pl.BlockSpec((1, _HID), lambda i: (0, 0)),
            pl.BlockSpec((1, _HID), lambda i: (0, 0)),
        ],
        out_specs=[
            pl.BlockSpec((1, 1, 1, _BN), lambda i: (i // _NB, i % _NB, 0, 0)),
            pl.BlockSpec((1, _BN, _REP_DIM), lambda i: (i // _NB, i % _NB, 0)),
        ],
        out_shape=[
            jax.ShapeDtypeStruct((_B, _NB, 1, _BN), jnp.float32),
            jax.ShapeDtypeStruct((_B, _N, _REP_DIM), jnp.float32),
        ],
    )(reps, W1, b1.reshape(1, _HID), W2)
    return lg.reshape(_B, _N), base


def _topk_body(lg_ref, b2_ref, g_ref, madd_ref, ind_ref, oh_ref):
    logits = lg_ref[...] + b2_ref[...]
    z = ((logits + g_ref[...]) + madd_ref[...]) / _TEMP
    zmax = jnp.max(z, axis=1, keepdims=True)
    e = jnp.exp(z - zmax)
    y = e / jnp.sum(e, axis=1, keepdims=True)
    col = lax.broadcasted_iota(jnp.int32, (_B, _N), 1)
    y = y + jnp.where(col == 0, jnp.float32(10000.0), jnp.float32(0.0))

    oh = jnp.zeros((_B, _N), jnp.int32)
    ind_cols = []
    for _ in range(_TOPK):
        m = jnp.max(y, axis=1, keepdims=True)
        idx = jnp.min(jnp.where(y == m, col, _N), axis=1, keepdims=True)
        sel = col == idx
        oh = oh | sel.astype(jnp.int32)
        y = jnp.where(sel, jnp.float32(-1.0), y)
        ind_cols.append(idx)
    ind_ref[...] = jnp.concatenate(ind_cols, axis=1)
    oh_ref[...] = oh


def _topk(lg, b2, g, madd):
    return pl.pallas_call(
        _topk_body,
        out_shape=[
            jax.ShapeDtypeStruct((_B, _TOPK), jnp.int32),
            jax.ShapeDtypeStruct((_B, _N), jnp.int32),
        ],
    )(lg, b2.reshape(1, 1), g, madd)


def _sc_scatter_body(base_ref, reps_ref, ind_ref, out_ref, idx_v, rows_v, sem):
    del base_ref
    b = lax.axis_index("s") * 2 + lax.axis_index("c")
    pltpu.sync_copy(ind_ref.at[b], idx_v)
    row0 = b * _N
    for k in range(_TOPK // 16):
        sl = pl.ds(k * 16, 16)
        idx_v[sl] = idx_v[sl] + jnp.full((16,), row0, jnp.int32)
    pltpu.async_copy(reps_ref.at[idx_v], rows_v, sem).wait()
    pltpu.async_copy(rows_v, out_ref.at[idx_v], sem).wait()


def _sc_scatter(base, reps_flat, ind):
    mesh = plsc.VectorSubcoreMesh(core_axis_name="c", subcore_axis_name="s")
    fn = _mpmd._mpmd_map(
        [(mesh, _sc_scatter_body)],
        jax.ShapeDtypeStruct((_B * _N, _REP_DIM), jnp.float32),
        input_output_aliases={0: 0},
        scratch_types=[
            pltpu.VMEM((_TOPK,), jnp.int32),
            pltpu.VMEM((_TOPK, _REP_DIM), jnp.float32),
            pltpu.SemaphoreType.DMA,
        ],
    )
    return fn(base, reps_flat, ind)


def _gumbel_noise(shape, eps=1e-20):
    U = jax.random.uniform(jax.random.key(42), shape, dtype=jnp.float32)
    return -jnp.log(-jnp.log(U + eps) + eps)


def kernel(reps, mask, W1, b1, W2, b2):
    lg, base = _compute_logits_and_base(reps, W1, b1, W2)
    madd = (~mask).astype(jnp.float32) * -10000.0
    ind, oh = _topk(lg, b2, _gumbel_noise((_B, _N)), madd)
    out = _sc_scatter(
        base.reshape(_B * _N, _REP_DIM),
        reps.reshape(_B * _N, _REP_DIM),
        ind,
    )
    sampled_reps = out.reshape(_B, _N, _REP_DIM)
    sampled_mask = oh.astype(bool) & mask
    return sampled_reps, sampled_mask, ind


# speedup vs baseline: 1.0806x; 1.0806x over previous
import jax
import jax.numpy as jnp
from jax import lax
from jax.experimental import pallas as pl
from jax.experimental.pallas import tpu as pltpu
from jax.experimental.pallas import tpu_sc as plsc
from jax._src.pallas import mpmd as _mpmd

_REP_DIM = 768
_HID = 384
_TOPK = 64
_TEMP = 0.1
_B, _N = 32, 2048
_BN = 1024
_NB = _N // _BN


def _logits_body(reps_ref, w1_ref, b1_ref, w2_ref, lg_ref, base_ref):
    x = reps_ref[0]
    h = jax.lax.dot_general(
        x, w1_ref[...], (((1,), (1,)), ((), ())),
        preferred_element_type=jnp.float32,
    )
    h = jnp.tanh(h + b1_ref[...])
    lg128 = jax.lax.dot_general(
        h.astype(jnp.bfloat16), w2_ref[...], (((1,), (0,)), ((), ())),
        preferred_element_type=jnp.float32,
    )
    lg_ref[0, 0, 0, :] = lg128[:, 0]
    base_ref[0] = jnp.zeros((_BN, _REP_DIM), jnp.float32)


def _compute_logits_and_base(reps, W1, b1, W2):
    call = pl.pallas_call(
        _logits_body,
        grid=(_B * _N // _BN,),
        in_specs=[
            pl.BlockSpec((1, _BN, _REP_DIM), lambda i: (i // _NB, i % _NB, 0)),
            pl.BlockSpec((_HID, _REP_DIM), lambda i: (0, 0)),
            pl.BlockSpec((1, _HID), lambda i: (0, 0)),
            pl.BlockSpec((_HID, 128), lambda i: (0, 0)),
        ],
        out_specs=[
            pl.BlockSpec((1, 1, 1, _BN), lambda i: (i // _NB, i % _NB, 0, 0)),
            pl.BlockSpec((1, _BN, _REP_DIM), lambda i: (i // _NB, i % _NB, 0)),
        ],
        out_shape=[
            jax.ShapeDtypeStruct((_B, _NB, 1, _BN), jnp.float32),
            jax.ShapeDtypeStruct((_B, _N, _REP_DIM), jnp.float32),
        ],
    )
    w2col = jnp.tile(W2.astype(jnp.bfloat16).reshape(_HID, 1), (1, 128))
    lg, base = call(reps, W1, b1.reshape(1, _HID), w2col)
    return lg.reshape(_B, _N), base


def _topk_body(lg_ref, b2_ref, g_ref, madd_ref, ind_ref, oh_ref):
    logits = lg_ref[...] + b2_ref[...]
    z = ((logits + g_ref[...]) + madd_ref[...]) / _TEMP
    zmax = jnp.max(z, axis=1, keepdims=True)
    e = jnp.exp(z - zmax)
    y = e / jnp.sum(e, axis=1, keepdims=True)
    col = lax.broadcasted_iota(jnp.int32, (_B, _N), 1)
    y = y + jnp.where(col == 0, jnp.float32(10000.0), jnp.float32(0.0))

    oh = jnp.zeros((_B, _N), jnp.int32)
    ind_cols = []
    for _ in range(_TOPK):
        m = jnp.max(y, axis=1, keepdims=True)
        idx = jnp.min(jnp.where(y == m, col, _N), axis=1, keepdims=True)
        sel = col == idx
        oh = oh | sel.astype(jnp.int32)
        y = jnp.where(sel, jnp.float32(-1.0), y)
        ind_cols.append(idx)
    ind_ref[...] = jnp.concatenate(ind_cols, axis=1)
    oh_ref[...] = oh


def _topk(lg, b2, g, madd):
    return pl.pallas_call(
        _topk_body,
        out_shape=[
            jax.ShapeDtypeStruct((_B, _TOPK), jnp.int32),
            jax.ShapeDtypeStruct((_B, _N), jnp.int32),
        ],
    )(lg, b2.reshape(1, 1), g, madd)


def _sc_scatter_body(base_ref, reps_ref, ind_ref, out_ref, idx_v, rows_v, sem):
    del base_ref
    b = lax.axis_index("s") * 2 + lax.axis_index("c")
    pltpu.sync_copy(ind_ref.at[b], idx_v)
    row0 = b * _N
    for k in range(_TOPK // 16):
        sl = pl.ds(k * 16, 16)
        idx_v[sl] = idx_v[sl] + jnp.full((16,), row0, jnp.int32)
    pltpu.async_copy(reps_ref.at[idx_v], rows_v, sem).wait()
    pltpu.async_copy(rows_v, out_ref.at[idx_v], sem).wait()


def _sc_scatter(base, reps_flat, ind):
    mesh = plsc.VectorSubcoreMesh(core_axis_name="c", subcore_axis_name="s")
    fn = _mpmd._mpmd_map(
        [(mesh, _sc_scatter_body)],
        jax.ShapeDtypeStruct((_B * _N, _REP_DIM), jnp.float32),
        input_output_aliases={0: 0},
        scratch_types=[
            pltpu.VMEM((_TOPK,), jnp.int32),
            pltpu.VMEM((_TOPK, _REP_DIM), jnp.float32),
            pltpu.SemaphoreType.DMA,
        ],
    )
    return fn(base, reps_flat, ind)


def _gumbel_noise(shape, eps=1e-20):
    U = jax.random.uniform(jax.random.key(42), shape, dtype=jnp.float32)
    return -jnp.log(-jnp.log(U + eps) + eps)


def kernel(reps, mask, W1, b1, W2, b2):
    lg, base = _compute_logits_and_base(reps, W1, b1, W2)
    madd = (~mask).astype(jnp.float32) * -10000.0
    ind, oh = _topk(lg, b2, _gumbel_noise((_B, _N)), madd)
    out = _sc_scatter(
        base.reshape(_B * _N, _REP_DIM),
        reps.reshape(_B * _N, _REP_DIM),
        ind,
    )
    sampled_reps = out.reshape(_B, _N, _REP_DIM)
    sampled_mask = oh.astype(bool) & mask
    return sampled_reps, sampled_mask, ind
